# baseline (speedup 1.0000x reference)
import functools

import jax
import jax.numpy as jnp
from jax import lax
from jax.experimental import pallas as pl
from jax.experimental.pallas import tpu as pltpu
from jax.experimental.pallas import tpu_sc as plsc

NR_CONCEPTS = 1000000
CONCEPT_DIM = 32
BATCH = 4096
HIST = 200
NB = BATCH * HIST

GB = 16
HB = 40
CHUNK = GB * HB
N_G = 128 // GB
N_H = HIST // HB
N_CHUNKS = N_G * N_H


def kernel(x, weight):
    info = plsc.get_sparse_core_info()
    nw = info.num_cores * info.num_subcores
    b_per_w = NB // nw

    mesh = plsc.VectorSubcoreMesh(core_axis_name="c", subcore_axis_name="s")

    @functools.partial(
        pl.kernel,
        mesh=mesh,
        out_type=jax.ShapeDtypeStruct((HIST, 4, 32, 8, 128), jnp.float32),
        scratch_types=(
            [pltpu.VMEM((CHUNK,), jnp.int32)] * 2
            + [pltpu.VMEM((CHUNK, CONCEPT_DIM), jnp.float32)] * 2
            + [pltpu.VMEM((HB, 4, 8, GB + 1), jnp.float32)] * 2
            + [pltpu.SemaphoreType.DMA] * 6
        ),
        compiler_params=pltpu.CompilerParams(use_tc_tiling_on_sc=False,
                                             needs_layout_passes=False),
    )
    def emb_kernel(idx_hbm, table_hbm, out_hbm,
                   idx0, idx1, buf0, buf1, sl0, sl1,
                   isem0, isem1, gsem0, gsem1, wsem0, wsem1):
        cid = lax.axis_index("c")
        sid = lax.axis_index("s")
        wid = sid * info.num_cores + cid
        base = wid * b_per_w

        idxb = (idx0, idx1)
        buf = (buf0, buf1)
        slab = (sl0, sl1)
        isem = (isem0, isem1)
        gsem = (gsem0, gsem1)
        wsem = (wsem0, wsem1)

        j16 = lax.iota(jnp.int32, 16)
        tr_half = (j16 // 8, j16 // 8 + 2)
        r_half = (j16 % 8, j16 % 8)

        def load_idx(s, c):
            g = c // N_H
            hb = c - g * N_H
            th0 = hb * (HB // 8)
            l0 = g * GB
            for th in range(HB // 8):
                for r in range(8):
                    off = (th0 + th) * 32768 + wid * 1024 + r * 128 + l0
                    pltpu.async_copy(
                        idx_hbm.at[pl.ds(off, GB)],
                        idxb[s].at[pl.ds((th * 8 + r) * GB, GB)], isem[s])
            pltpu.make_async_copy(
                idx_hbm.at[pl.ds(0, CHUNK)], idxb[s], isem[s]).wait()

        def start_gather(s):
            return pltpu.async_copy(table_hbm.at[idxb[s]], buf[s], gsem[s])

        def wait_gather(s):
            pltpu.make_async_copy(table_hbm.at[idxb[s]], buf[s],
                                  gsem[s]).wait()

        def transpose(s):
            bs = buf[s]
            sls = slab[s]

            for bi in range(GB):
                biv = jnp.full((16,), bi, jnp.int32)

                @plsc.parallel_loop(0, HB, unroll=4)
                def h_body(h, bi=bi, biv=biv):
                    row = h * GB + bi
                    hv = jnp.full((16,), h, jnp.int32)
                    for k in range(2):
                        v = bs[row, pl.ds(16 * k, 16)]
                        plsc.store_scatter(
                            sls, [hv, tr_half[k], r_half[k], biv], v)

        def start_write(s, c):
            g = c // N_H
            hb = c - g * N_H
            h0 = hb * HB
            l0 = g * GB
            return pltpu.async_copy(
                slab[s].at[:, :, :, pl.ds(0, GB)],
                out_hbm.at[pl.ds(h0, HB), slice(None), wid, slice(None),
                           pl.ds(l0, GB)],
                wsem[s])

        def wait_write(s, c):
            g = c // N_H
            hb = c - g * N_H
            h0 = hb * HB
            l0 = g * GB
            pltpu.make_async_copy(
                slab[s].at[:, :, :, pl.ds(0, GB)],
                out_hbm.at[pl.ds(h0, HB), slice(None), wid, slice(None),
                           pl.ds(l0, GB)],
                wsem[s]).wait()

        for s in (0, 1):
            load_idx(s, s)
            start_gather(s)

        def step(c, first, last):
            for s in (0, 1):
                cc = c + s
                if not first:
                    wait_write(s, cc - 2)
                wait_gather(s)
                transpose(s)
                start_write(s, cc)
                if not last:
                    load_idx(s, cc + 2)
                    start_gather(s)

        step(0, True, False)

        def body(k, carry):
            step(2 * k, False, False)
            return carry

        lax.fori_loop(1, N_CHUNKS // 2 - 1, body, 0)
        step(N_CHUNKS - 2, False, True)

        wait_write(0, N_CHUNKS - 2)
        wait_write(1, N_CHUNKS - 1)

    xt = x.reshape(32, 128, 25, 8).transpose(2, 0, 3, 1).reshape(NB)
    out5 = emb_kernel(xt, weight)
    return out5.transpose(2, 4, 0, 1, 3).reshape(BATCH, HIST, CONCEPT_DIM)


# speedup vs baseline: 1.0033x; 1.0033x over previous
import functools

import jax
import jax.numpy as jnp
from jax import lax
from jax.experimental import pallas as pl
from jax.experimental.pallas import tpu as pltpu
from jax.experimental.pallas import tpu_sc as plsc

NR_CONCEPTS = 1000000
CONCEPT_DIM = 32
BATCH = 4096
HIST = 200
NB = BATCH * HIST

GB = 16
HB = 40
CHUNK = GB * HB
N_G = 128 // GB
N_H = HIST // HB
N_CHUNKS = N_G * N_H


def kernel(x, weight):
    info = plsc.get_sparse_core_info()
    nw = info.num_cores * info.num_subcores
    b_per_w = NB // nw

    mesh = plsc.VectorSubcoreMesh(core_axis_name="c", subcore_axis_name="s")

    @functools.partial(
        pl.kernel,
        mesh=mesh,
        out_type=jax.ShapeDtypeStruct((HIST, 4, 32, 8, 128), jnp.float32),
        scratch_types=(
            [pltpu.VMEM((CHUNK,), jnp.int32)] * 2
            + [pltpu.VMEM((CHUNK, CONCEPT_DIM), jnp.float32)] * 2
            + [pltpu.VMEM((HB, 4, 8, GB + 1), jnp.float32)] * 2
            + [pltpu.SemaphoreType.DMA] * 6
        ),
        compiler_params=pltpu.CompilerParams(use_tc_tiling_on_sc=False,
                                             needs_layout_passes=False),
    )
    def emb_kernel(idx_hbm, table_hbm, out_hbm,
                   idx0, idx1, buf0, buf1, sl0, sl1,
                   isem0, isem1, gsem0, gsem1, wsem0, wsem1):
        cid = lax.axis_index("c")
        sid = lax.axis_index("s")
        wid = sid * info.num_cores + cid
        base = wid * b_per_w

        idxb = (idx0, idx1)
        buf = (buf0, buf1)
        slab = (sl0, sl1)
        isem = (isem0, isem1)
        gsem = (gsem0, gsem1)
        wsem = (wsem0, wsem1)

        j16 = lax.iota(jnp.int32, 16)
        tr_half = (j16 // 8, j16 // 8 + 2)
        r_half = (j16 % 8, j16 % 8)

        def load_idx(s, c):
            g = c // N_H
            hb = c - g * N_H
            off0 = base + g * (GB * HIST) + hb * HB
            ds = []
            for bi in range(GB):
                ds.append(pltpu.async_copy(
                    idx_hbm.at[pl.ds(off0 + bi * HIST, HB)],
                    idxb[s].at[pl.ds(bi * HB, HB)], isem[s]))
            for d in ds:
                d.wait()

        def start_gather(s):
            return pltpu.async_copy(table_hbm.at[idxb[s]], buf[s], gsem[s])

        def wait_gather(s):
            pltpu.make_async_copy(table_hbm.at[idxb[s]], buf[s],
                                  gsem[s]).wait()

        def transpose(s):
            bs = buf[s]
            sls = slab[s]

            for bi in range(GB):
                biv = jnp.full((16,), bi, jnp.int32)

                @plsc.parallel_loop(0, HB, unroll=8)
                def h_body(h, bi=bi, biv=biv):
                    row = bi * HB + h
                    hv = jnp.full((16,), h, jnp.int32)
                    for k in range(2):
                        v = bs[row, pl.ds(16 * k, 16)]
                        plsc.store_scatter(
                            sls, [hv, tr_half[k], r_half[k], biv], v)

        def start_write(s, c):
            g = c // N_H
            hb = c - g * N_H
            h0 = hb * HB
            l0 = g * GB
            return pltpu.async_copy(
                slab[s].at[:, :, :, pl.ds(0, GB)],
                out_hbm.at[pl.ds(h0, HB), slice(None), wid, slice(None),
                           pl.ds(l0, GB)],
                wsem[s])

        def wait_write(s, c):
            g = c // N_H
            hb = c - g * N_H
            h0 = hb * HB
            l0 = g * GB
            pltpu.make_async_copy(
                slab[s].at[:, :, :, pl.ds(0, GB)],
                out_hbm.at[pl.ds(h0, HB), slice(None), wid, slice(None),
                           pl.ds(l0, GB)],
                wsem[s]).wait()

        for s in (0, 1):
            load_idx(s, s)
            start_gather(s)

        def step(c, first, last):
            for s in (0, 1):
                cc = c + s
                if not first:
                    wait_write(s, cc - 2)
                wait_gather(s)
                transpose(s)
                start_write(s, cc)
                if not last:
                    load_idx(s, cc + 2)
                    start_gather(s)

        step(0, True, False)

        def body(k, carry):
            step(2 * k, False, False)
            return carry

        lax.fori_loop(1, N_CHUNKS // 2 - 1, body, 0)
        step(N_CHUNKS - 2, False, True)

        wait_write(0, N_CHUNKS - 2)
        wait_write(1, N_CHUNKS - 1)

    out5 = emb_kernel(x.reshape(NB), weight)
    return out5.transpose(2, 4, 0, 1, 3).reshape(BATCH, HIST, CONCEPT_DIM)


# speedup vs baseline: 1.0079x; 1.0047x over previous
import functools

import jax
import jax.numpy as jnp
from jax import lax
from jax.experimental import pallas as pl
from jax.experimental.pallas import tpu as pltpu
from jax.experimental.pallas import tpu_sc as plsc

NR_CONCEPTS = 1000000
CONCEPT_DIM = 32
BATCH = 4096
HIST = 200
NB = BATCH * HIST

GB = 16
HB = 40
CHUNK = GB * HB
N_G = 128 // GB
N_H = HIST // HB
N_CHUNKS = N_G * N_H


def kernel(x, weight):
    info = plsc.get_sparse_core_info()
    nw = info.num_cores * info.num_subcores
    b_per_w = NB // nw

    mesh = plsc.VectorSubcoreMesh(core_axis_name="c", subcore_axis_name="s")

    @functools.partial(
        pl.kernel,
        mesh=mesh,
        out_type=jax.ShapeDtypeStruct((HIST, 4, 32, 8, 128), jnp.float32),
        scratch_types=(
            [pltpu.VMEM((CHUNK,), jnp.int32)] * 2
            + [pltpu.VMEM((CHUNK, CONCEPT_DIM), jnp.float32)] * 2
            + [pltpu.VMEM((HB, 4, 8, GB + 1), jnp.float32)] * 2
            + [pltpu.SemaphoreType.DMA] * 6
        ),
        compiler_params=pltpu.CompilerParams(use_tc_tiling_on_sc=False,
                                             needs_layout_passes=False),
    )
    def emb_kernel(idx_hbm, table_hbm, out_hbm,
                   idx0, idx1, buf0, buf1, sl0, sl1,
                   isem0, isem1, gsem0, gsem1, wsem0, wsem1):
        cid = lax.axis_index("c")
        sid = lax.axis_index("s")
        wid = sid * info.num_cores + cid
        base = wid * b_per_w

        idxb = (idx0, idx1)
        buf = (buf0, buf1)
        slab = (sl0, sl1)
        isem = (isem0, isem1)
        gsem = (gsem0, gsem1)
        wsem = (wsem0, wsem1)

        j16 = lax.iota(jnp.int32, 16)
        tr_half = (j16 // 8, j16 // 8 + 2)
        r_half = (j16 % 8, j16 % 8)

        def load_idx(s, c):
            g = c // N_H
            hb = c - g * N_H
            off0 = base + g * (GB * HIST) + hb * HB
            ds = []
            for bi in range(GB):
                ds.append(pltpu.async_copy(
                    idx_hbm.at[pl.ds(off0 + bi * HIST, HB)],
                    idxb[s].at[pl.ds(bi * HB, HB)], isem[s]))
            for d in ds:
                d.wait()

        def start_gather(s):
            return pltpu.async_copy(table_hbm.at[idxb[s]], buf[s], gsem[s])

        def wait_gather(s):
            pltpu.make_async_copy(table_hbm.at[idxb[s]], buf[s],
                                  gsem[s]).wait()

        def transpose(s):
            bs = buf[s]
            sls = slab[s]

            for bi in range(GB):
                biv = jnp.full((16,), bi, jnp.int32)

                @plsc.parallel_loop(0, HB, unroll=4)
                def h_body(h, bi=bi, biv=biv):
                    row = bi * HB + h
                    hv = jnp.full((16,), h, jnp.int32)
                    for k in range(2):
                        v = bs[row, pl.ds(16 * k, 16)]
                        plsc.store_scatter(
                            sls, [hv, tr_half[k], r_half[k], biv], v)

        def start_write(s, c):
            g = c // N_H
            hb = c - g * N_H
            h0 = hb * HB
            l0 = g * GB
            return pltpu.async_copy(
                slab[s].at[:, :, :, pl.ds(0, GB)],
                out_hbm.at[pl.ds(h0, HB), slice(None), wid, slice(None),
                           pl.ds(l0, GB)],
                wsem[s])

        def wait_write(s, c):
            g = c // N_H
            hb = c - g * N_H
            h0 = hb * HB
            l0 = g * GB
            pltpu.make_async_copy(
                slab[s].at[:, :, :, pl.ds(0, GB)],
                out_hbm.at[pl.ds(h0, HB), slice(None), wid, slice(None),
                           pl.ds(l0, GB)],
                wsem[s]).wait()

        for s in (0, 1):
            load_idx(s, s)
            start_gather(s)

        def step(c, first, last):
            for s in (0, 1):
                cc = c + s
                if not first:
                    wait_write(s, cc - 2)
                wait_gather(s)
                transpose(s)
                start_write(s, cc)
                if not last:
                    load_idx(s, cc + 2)
                    start_gather(s)

        step(0, True, False)

        def body(k, carry):
            step(2 * k, False, False)
            return carry

        lax.fori_loop(1, N_CHUNKS // 2 - 1, body, 0)
        step(N_CHUNKS - 2, False, True)

        wait_write(0, N_CHUNKS - 2)
        wait_write(1, N_CHUNKS - 1)

    out5 = emb_kernel(x.reshape(NB), weight)
    return out5.transpose(2, 4, 0, 1, 3).reshape(BATCH, HIST, CONCEPT_DIM)
